# Initial kernel scaffold; baseline (speedup 1.0000x reference)
import jax
import jax.numpy as jnp
from jax.experimental import pallas as pl


def kernel(x, edge_index, batch, Wl0, Wr0, b0, g0, be0, Wl1, Wr1, b1, g1, be1, Wl2, Wr2, b2, g2, be2, Wa, ba, Wb, bb):
    raise NotImplementedError("write your pallas kernel here")


# speedup vs baseline: 4.8695x; 4.8695x over previous
import functools

import jax
import jax.numpy as jnp
from jax import lax
from jax.experimental import pallas as pl
from jax.experimental.pallas import tpu as pltpu
from jax.experimental.pallas import tpu_sc as plsc

N = 10000
E = 320000
D = 128
H = 64
G = 64
EPS = 1e-5

W0 = 80

NCORES = 2
NSUB = 16
NW = NCORES * NSUB
EPW = E // NW
CH = 80
NCH = EPW // CH
RMAIN = 624
RST = 104

_PREC = lax.Precision.HIGHEST


def _make_sc_agg(W):
    mesh = plsc.VectorSubcoreMesh(core_axis_name="c", subcore_axis_name="s")

    @functools.partial(
        pl.kernel,
        mesh=mesh,
        out_type=jax.ShapeDtypeStruct((NCORES, N, W), jnp.float32),
        scratch_types=[
            pltpu.VMEM((CH,), jnp.int32),
            pltpu.VMEM((CH,), jnp.int32),
            pltpu.VMEM((CH, W), jnp.float32),
            pltpu.VMEM((8, W), jnp.float32),
            pltpu.VMEM((RST, W), jnp.float32),
            pltpu.VMEM_SHARED((N, W), jnp.float32),
            pltpu.SemaphoreType.DMA,
        ],
        compiler_params=pltpu.CompilerParams(use_tc_tiling_on_sc=False),
    )
    def agg(p_hbm, src_hbm, dst_hbm, out_hbm, sidx, didx, rows, zbuf, wb, A, sem):
        cid = lax.axis_index("c")
        sid = lax.axis_index("s")
        wid = cid * NSUB + sid
        r0 = pl.multiple_of(sid * RMAIN, 8)
        t0 = pl.multiple_of(NSUB * RMAIN + sid * 8, 8)

        zeros16 = jnp.zeros((16,), jnp.float32)
        for r in range(8):
            for c in range(W // 16):
                zbuf[r, pl.ds(c * 16, 16)] = zeros16

        def zbody(j, carry):
            pltpu.sync_copy(zbuf, A.at[pl.ds(pl.multiple_of(r0 + j * 8, 8), 8)])
            return carry

        lax.fori_loop(0, RMAIN // 8, zbody, 0)

        @pl.when(sid < (N - NSUB * RMAIN) // 8)
        def _zero_tail():
            pltpu.sync_copy(zbuf, A.at[pl.ds(t0, 8)])

        plsc.subcore_barrier()

        e0 = wid * EPW

        def ebody(i, carry):
            off = pl.multiple_of(e0 + i * CH, 16)
            pltpu.sync_copy(src_hbm.at[pl.ds(off, CH)], sidx)
            pltpu.sync_copy(dst_hbm.at[pl.ds(off, CH)], didx)
            pltpu.async_copy(p_hbm.at[sidx], rows, sem).wait()
            pltpu.sync_copy(rows, A.at[didx], add=True)
            return carry

        lax.fori_loop(0, NCH, ebody, 0)

        plsc.subcore_barrier()

        def obody(j, carry):
            o = pl.multiple_of(r0 + j * RST, 8)
            pltpu.sync_copy(A.at[pl.ds(o, RST)], wb)
            pltpu.sync_copy(wb, out_hbm.at[cid, pl.ds(o, RST)])
            return carry

        lax.fori_loop(0, RMAIN // RST, obody, 0)

        @pl.when(sid < (N - NSUB * RMAIN) // 8)
        def _write_tail():
            pltpu.sync_copy(A.at[pl.ds(t0, 8)], zbuf)
            pltpu.sync_copy(zbuf, out_hbm.at[cid, pl.ds(t0, 8)])

    return agg


_sc_cache = {}


def _sc_agg80(p, src, dst):
    if W0 not in _sc_cache:
        _sc_cache[W0] = _make_sc_agg(W0)
    return _sc_cache[W0](p, src, dst)


def _pad_table(p):
    ones = jnp.ones((N, 1), jnp.float32)
    zeros = jnp.zeros((N, W0 - H - 1), jnp.float32)
    return jnp.concatenate([p, ones, zeros], axis=1)


def _tc0_body(x_ref, wl_ref, wr_ref, b_ref, p_ref, q_ref):
    x = x_ref[...]
    p_ref[...] = _pad_table(jnp.dot(x, wl_ref[...], precision=_PREC))
    q_ref[...] = jnp.dot(x, wr_ref[...], precision=_PREC) + b_ref[...]


def _tc0(x, wl, wr, b):
    return pl.pallas_call(
        _tc0_body,
        out_shape=(
            jax.ShapeDtypeStruct((N, W0), jnp.float32),
            jax.ShapeDtypeStruct((N, H), jnp.float32),
        ),
    )(x, wl, wr, b)


def _bn_relu(h, g, be):
    mu = jnp.mean(h, axis=0, keepdims=True)
    var = jnp.mean((h - mu) * (h - mu), axis=0, keepdims=True)
    return jnp.maximum(g * (h - mu) / jnp.sqrt(var + EPS) + be, 0.0)


def _tc1_body(agg_ref, q_ref, g_ref, be_ref, wl_ref, wr_ref, b_ref,
              p_ref, qn_ref, cnt_ref):
    a = agg_ref[0] + agg_ref[1]
    cnt = jnp.maximum(a[:, H:H + 1], 1.0)
    h = a[:, :H] / cnt + q_ref[...]
    h = _bn_relu(h, g_ref[...], be_ref[...])
    p_ref[...] = _pad_table(jnp.dot(h, wl_ref[...], precision=_PREC))
    qn_ref[...] = jnp.dot(h, wr_ref[...], precision=_PREC) + b_ref[...]
    cnt_ref[...] = cnt


def _tc1(agg, q, g, be, wl, wr, b):
    return pl.pallas_call(
        _tc1_body,
        out_shape=(
            jax.ShapeDtypeStruct((N, W0), jnp.float32),
            jax.ShapeDtypeStruct((N, H), jnp.float32),
            jax.ShapeDtypeStruct((N, 1), jnp.float32),
        ),
    )(agg, q, g, be, wl, wr, b)


def _tc2_body(agg_ref, cnt_ref, q_ref, g_ref, be_ref, wl_ref, wr_ref, b_ref,
              p_ref, qn_ref):
    a = agg_ref[0] + agg_ref[1]
    h = a[:, :H] / cnt_ref[...] + q_ref[...]
    h = _bn_relu(h, g_ref[...], be_ref[...])
    p_ref[...] = _pad_table(jnp.dot(h, wl_ref[...], precision=_PREC))
    qn_ref[...] = jnp.dot(h, wr_ref[...], precision=_PREC) + b_ref[...]


def _tc2(agg, cnt, q, g, be, wl, wr, b):
    return pl.pallas_call(
        _tc2_body,
        out_shape=(
            jax.ShapeDtypeStruct((N, W0), jnp.float32),
            jax.ShapeDtypeStruct((N, H), jnp.float32),
        ),
    )(agg, cnt, q, g, be, wl, wr, b)


def _tc3_body(agg_ref, cnt_ref, q_ref, g_ref, be_ref, batch_ref,
              wa_ref, ba_ref, wb_ref, bb_ref, out_ref):
    a = agg_ref[0] + agg_ref[1]
    h = a[:, :H] / cnt_ref[...] + q_ref[...]
    h = _bn_relu(h, g_ref[...], be_ref[...])
    gid = lax.broadcasted_iota(jnp.int32, (1, G), 1)
    m = (batch_ref[...] == gid).astype(jnp.float32)
    sums = lax.dot_general(m, h, (((0,), (0,)), ((), ())),
                           precision=_PREC)
    ones = jnp.ones((N, 1), jnp.float32)
    pcnt = lax.dot_general(m, ones, (((0,), (0,)), ((), ())),
                           precision=_PREC)
    emb = sums / jnp.maximum(pcnt, 1.0)
    z = jnp.maximum(jnp.dot(emb, wa_ref[...], precision=_PREC) + ba_ref[...],
                    0.0)
    out_ref[...] = jnp.dot(z, wb_ref[...], precision=_PREC) + bb_ref[...]


def _tc3(agg, cnt, q, g, be, batch2, wa, ba, wb, bb):
    return pl.pallas_call(
        _tc3_body,
        out_shape=jax.ShapeDtypeStruct((G, 2), jnp.float32),
    )(agg, cnt, q, g, be, batch2, wa, ba, wb, bb)


def kernel(x, edge_index, batch, Wl0, Wr0, b0, g0, be0, Wl1, Wr1, b1, g1, be1,
           Wl2, Wr2, b2, g2, be2, Wa, ba, Wb, bb):
    src = edge_index[0]
    dst = edge_index[1]
    batch2 = batch.reshape(N, 1)
    b0r, g0r, be0r = b0.reshape(1, H), g0.reshape(1, H), be0.reshape(1, H)
    b1r, g1r, be1r = b1.reshape(1, H), g1.reshape(1, H), be1.reshape(1, H)
    b2r, g2r, be2r = b2.reshape(1, H), g2.reshape(1, H), be2.reshape(1, H)
    bar = ba.reshape(1, H // 2)
    bbr = bb.reshape(1, 2)

    p0, q0 = _tc0(x, Wl0, Wr0, b0r)
    agg0 = _sc_agg80(p0, src, dst)
    p1, q1, cnt = _tc1(agg0, q0, g0r, be0r, Wl1, Wr1, b1r)
    agg1 = _sc_agg80(p1, src, dst)
    p2, q2 = _tc2(agg1, cnt, q1, g1r, be1r, Wl2, Wr2, b2r)
    agg2 = _sc_agg80(p2, src, dst)
    out = _tc3(agg2, cnt, q2, g2r, be2r, batch2, Wa, bar, Wb, bbr)
    return out


# speedup vs baseline: 9.0887x; 1.8664x over previous
import functools

import jax
import jax.numpy as jnp
from jax import lax
from jax.experimental import pallas as pl
from jax.experimental.pallas import tpu as pltpu
from jax.experimental.pallas import tpu_sc as plsc

N = 10000
E = 320000
D = 128
H = 64
G = 64
EPS = 1e-5

W0 = 80

NCORES = 2
NSUB = 16
NW = NCORES * NSUB
EPW = E // NW
CH = 100
NCH = EPW // CH
RMAIN = 624
RST = 104

_PREC = lax.Precision.HIGHEST


def _make_sc_agg(W):
    mesh = plsc.VectorSubcoreMesh(core_axis_name="c", subcore_axis_name="s")

    @functools.partial(
        pl.kernel,
        mesh=mesh,
        out_type=jax.ShapeDtypeStruct((NCORES, N, W), jnp.float32),
        scratch_types=[
            pltpu.VMEM((NCH, CH), jnp.int32),
            pltpu.VMEM((NCH, CH), jnp.int32),
            pltpu.VMEM((CH, W), jnp.float32),
            pltpu.VMEM((CH, W), jnp.float32),
            pltpu.VMEM((8, W), jnp.float32),
            pltpu.VMEM((RST, W), jnp.float32),
            pltpu.VMEM_SHARED((N, W), jnp.float32),
            pltpu.SemaphoreType.DMA,
            pltpu.SemaphoreType.DMA,
        ],
        compiler_params=pltpu.CompilerParams(use_tc_tiling_on_sc=False),
    )
    def agg(p_hbm, src_hbm, dst_hbm, out_hbm, sidx, didx, rows0, rows1,
            zbuf, wb, A, sem0, sem1):
        cid = lax.axis_index("c")
        sid = lax.axis_index("s")
        wid = cid * NSUB + sid
        r0 = pl.multiple_of(sid * RMAIN, 8)
        t0 = pl.multiple_of(NSUB * RMAIN + sid * 8, 8)

        pltpu.sync_copy(src_hbm.at[wid], sidx)
        pltpu.sync_copy(dst_hbm.at[wid], didx)

        zeros16 = jnp.zeros((16,), jnp.float32)
        for r in range(8):
            for c in range(W // 16):
                zbuf[r, pl.ds(c * 16, 16)] = zeros16

        def zbody(j, carry):
            pltpu.sync_copy(zbuf, A.at[pl.ds(pl.multiple_of(r0 + j * 8, 8), 8)])
            return carry

        lax.fori_loop(0, RMAIN // 8, zbody, 0)

        @pl.when(sid < (N - NSUB * RMAIN) // 8)
        def _zero_tail():
            pltpu.sync_copy(zbuf, A.at[pl.ds(t0, 8)])

        plsc.subcore_barrier()

        pltpu.async_copy(p_hbm.at[sidx.at[0]], rows0, sem0).wait()

        def ebody(k, carry):
            i = 2 * k
            d1 = pltpu.async_copy(p_hbm.at[sidx.at[i + 1]], rows1, sem1)
            pltpu.sync_copy(rows0, A.at[didx.at[i]], add=True)
            d1.wait()
            inext = lax.rem(i + 2, NCH)
            d0 = pltpu.async_copy(p_hbm.at[sidx.at[inext]], rows0, sem0)
            pltpu.sync_copy(rows1, A.at[didx.at[i + 1]], add=True)
            d0.wait()
            return carry

        lax.fori_loop(0, NCH // 2, ebody, 0)

        plsc.subcore_barrier()

        def obody(j, carry):
            o = pl.multiple_of(r0 + j * RST, 8)
            pltpu.sync_copy(A.at[pl.ds(o, RST)], wb)
            pltpu.sync_copy(wb, out_hbm.at[cid, pl.ds(o, RST)])
            return carry

        lax.fori_loop(0, RMAIN // RST, obody, 0)

        @pl.when(sid < (N - NSUB * RMAIN) // 8)
        def _write_tail():
            pltpu.sync_copy(A.at[pl.ds(t0, 8)], zbuf)
            pltpu.sync_copy(zbuf, out_hbm.at[cid, pl.ds(t0, 8)])

    return agg


_sc_cache = {}


def _sc_agg80(p, src, dst):
    if W0 not in _sc_cache:
        _sc_cache[W0] = _make_sc_agg(W0)
    return _sc_cache[W0](p, src, dst)


def _pad_table(p):
    ones = jnp.ones((N, 1), jnp.float32)
    zeros = jnp.zeros((N, W0 - H - 1), jnp.float32)
    return jnp.concatenate([p, ones, zeros], axis=1)


def _tc0_body(x_ref, wl_ref, wr_ref, b_ref, p_ref, q_ref):
    x = x_ref[...]
    p_ref[...] = _pad_table(jnp.dot(x, wl_ref[...], precision=_PREC))
    q_ref[...] = jnp.dot(x, wr_ref[...], precision=_PREC) + b_ref[...]


def _tc0(x, wl, wr, b):
    return pl.pallas_call(
        _tc0_body,
        out_shape=(
            jax.ShapeDtypeStruct((N, W0), jnp.float32),
            jax.ShapeDtypeStruct((N, H), jnp.float32),
        ),
    )(x, wl, wr, b)


def _bn_relu(h, g, be):
    mu = jnp.mean(h, axis=0, keepdims=True)
    var = jnp.mean((h - mu) * (h - mu), axis=0, keepdims=True)
    return jnp.maximum(g * (h - mu) / jnp.sqrt(var + EPS) + be, 0.0)


def _tc1_body(agg_ref, q_ref, g_ref, be_ref, wl_ref, wr_ref, b_ref,
              p_ref, qn_ref, cnt_ref):
    a = agg_ref[0] + agg_ref[1]
    cnt = jnp.maximum(a[:, H:H + 1], 1.0)
    h = a[:, :H] / cnt + q_ref[...]
    h = _bn_relu(h, g_ref[...], be_ref[...])
    p_ref[...] = _pad_table(jnp.dot(h, wl_ref[...], precision=_PREC))
    qn_ref[...] = jnp.dot(h, wr_ref[...], precision=_PREC) + b_ref[...]
    cnt_ref[...] = cnt


def _tc1(agg, q, g, be, wl, wr, b):
    return pl.pallas_call(
        _tc1_body,
        out_shape=(
            jax.ShapeDtypeStruct((N, W0), jnp.float32),
            jax.ShapeDtypeStruct((N, H), jnp.float32),
            jax.ShapeDtypeStruct((N, 1), jnp.float32),
        ),
    )(agg, q, g, be, wl, wr, b)


def _tc2_body(agg_ref, cnt_ref, q_ref, g_ref, be_ref, wl_ref, wr_ref, b_ref,
              p_ref, qn_ref):
    a = agg_ref[0] + agg_ref[1]
    h = a[:, :H] / cnt_ref[...] + q_ref[...]
    h = _bn_relu(h, g_ref[...], be_ref[...])
    p_ref[...] = _pad_table(jnp.dot(h, wl_ref[...], precision=_PREC))
    qn_ref[...] = jnp.dot(h, wr_ref[...], precision=_PREC) + b_ref[...]


def _tc2(agg, cnt, q, g, be, wl, wr, b):
    return pl.pallas_call(
        _tc2_body,
        out_shape=(
            jax.ShapeDtypeStruct((N, W0), jnp.float32),
            jax.ShapeDtypeStruct((N, H), jnp.float32),
        ),
    )(agg, cnt, q, g, be, wl, wr, b)


def _tc3_body(agg_ref, cnt_ref, q_ref, g_ref, be_ref, batch_ref,
              wa_ref, ba_ref, wb_ref, bb_ref, out_ref):
    a = agg_ref[0] + agg_ref[1]
    h = a[:, :H] / cnt_ref[...] + q_ref[...]
    h = _bn_relu(h, g_ref[...], be_ref[...])
    gid = lax.broadcasted_iota(jnp.int32, (1, G), 1)
    m = (batch_ref[...] == gid).astype(jnp.float32)
    sums = lax.dot_general(m, h, (((0,), (0,)), ((), ())),
                           precision=_PREC)
    ones = jnp.ones((N, 1), jnp.float32)
    pcnt = lax.dot_general(m, ones, (((0,), (0,)), ((), ())),
                           precision=_PREC)
    emb = sums / jnp.maximum(pcnt, 1.0)
    z = jnp.maximum(jnp.dot(emb, wa_ref[...], precision=_PREC) + ba_ref[...],
                    0.0)
    out_ref[...] = jnp.dot(z, wb_ref[...], precision=_PREC) + bb_ref[...]


def _tc3(agg, cnt, q, g, be, batch2, wa, ba, wb, bb):
    return pl.pallas_call(
        _tc3_body,
        out_shape=jax.ShapeDtypeStruct((G, 2), jnp.float32),
    )(agg, cnt, q, g, be, batch2, wa, ba, wb, bb)


def kernel(x, edge_index, batch, Wl0, Wr0, b0, g0, be0, Wl1, Wr1, b1, g1, be1,
           Wl2, Wr2, b2, g2, be2, Wa, ba, Wb, bb):
    src = edge_index[0].reshape(NW, NCH, CH)
    dst = edge_index[1].reshape(NW, NCH, CH)
    batch2 = batch.reshape(N, 1)
    b0r, g0r, be0r = b0.reshape(1, H), g0.reshape(1, H), be0.reshape(1, H)
    b1r, g1r, be1r = b1.reshape(1, H), g1.reshape(1, H), be1.reshape(1, H)
    b2r, g2r, be2r = b2.reshape(1, H), g2.reshape(1, H), be2.reshape(1, H)
    bar = ba.reshape(1, H // 2)
    bbr = bb.reshape(1, 2)

    p0, q0 = _tc0(x, Wl0, Wr0, b0r)
    agg0 = _sc_agg80(p0, src, dst)
    p1, q1, cnt = _tc1(agg0, q0, g0r, be0r, Wl1, Wr1, b1r)
    agg1 = _sc_agg80(p1, src, dst)
    p2, q2 = _tc2(agg1, cnt, q1, g1r, be1r, Wl2, Wr2, b2r)
    agg2 = _sc_agg80(p2, src, dst)
    out = _tc3(agg2, cnt, q2, g2r, be2r, batch2, Wa, bar, Wb, bbr)
    return out


# speedup vs baseline: 12.8981x; 1.4191x over previous
import functools

import jax
import jax.numpy as jnp
from jax import lax
from jax.experimental import pallas as pl
from jax.experimental.pallas import tpu as pltpu
from jax.experimental.pallas import tpu_sc as plsc

N = 10000
E = 320000
D = 128
H = 64
G = 64
EPS = 1e-5

W0 = 80

NCORES = 2
NSUB = 16
NW = NCORES * NSUB
EPW = E // NW
CH = 100
NCH = EPW // CH
RMAIN = 624
RST = 104

_PREC = lax.Precision.HIGHEST


def _make_sc_agg(W):
    mesh = plsc.VectorSubcoreMesh(core_axis_name="c", subcore_axis_name="s")

    @functools.partial(
        pl.kernel,
        mesh=mesh,
        out_type=jax.ShapeDtypeStruct((NCORES, N, W), jnp.float32),
        scratch_types=[
            pltpu.VMEM((NCH, CH), jnp.int32),
            pltpu.VMEM((NCH, CH), jnp.int32),
            pltpu.VMEM((4, CH, W), jnp.float32),
            pltpu.VMEM((8, W), jnp.float32),
            pltpu.VMEM((RST, W), jnp.float32),
            pltpu.VMEM_SHARED((N, W), jnp.float32),
            [pltpu.SemaphoreType.DMA] * 4,
        ],
        compiler_params=pltpu.CompilerParams(use_tc_tiling_on_sc=False),
    )
    def agg(p_hbm, src_hbm, dst_hbm, out_hbm, sidx, didx, rows,
            zbuf, wb, A, sems):
        cid = lax.axis_index("c")
        sid = lax.axis_index("s")
        wid = cid * NSUB + sid
        r0 = pl.multiple_of(sid * RMAIN, 8)
        t0 = pl.multiple_of(NSUB * RMAIN + sid * 8, 8)

        pltpu.sync_copy(src_hbm.at[wid], sidx)
        pltpu.sync_copy(dst_hbm.at[wid], didx)

        zeros16 = jnp.zeros((16,), jnp.float32)
        for r in range(8):
            for c in range(W // 16):
                zbuf[r, pl.ds(c * 16, 16)] = zeros16

        def zbody(j, carry):
            pltpu.sync_copy(zbuf, A.at[pl.ds(pl.multiple_of(r0 + j * 8, 8), 8)])
            return carry

        lax.fori_loop(0, RMAIN // 8, zbody, 0)

        @pl.when(sid < (N - NSUB * RMAIN) // 8)
        def _zero_tail():
            pltpu.sync_copy(zbuf, A.at[pl.ds(t0, 8)])

        plsc.subcore_barrier()

        for b in range(3):
            pltpu.async_copy(p_hbm.at[sidx.at[b]], rows.at[b], sems[b])

        def ebody(k, carry):
            i0 = 4 * k
            for b in range(4):
                i = i0 + b
                inext = lax.rem(i + 3, NCH)
                pltpu.async_copy(p_hbm.at[sidx.at[inext]],
                                 rows.at[(b + 3) % 4], sems[(b + 3) % 4])
                pltpu.make_async_copy(p_hbm.at[sidx.at[i]],
                                      rows.at[b], sems[b]).wait()
                pltpu.sync_copy(rows.at[b], A.at[didx.at[i]], add=True)
            return carry

        lax.fori_loop(0, NCH // 4, ebody, 0)

        for b in range(3):
            pltpu.make_async_copy(p_hbm.at[sidx.at[b]], rows.at[b],
                                  sems[b]).wait()

        plsc.subcore_barrier()

        def obody(j, carry):
            o = pl.multiple_of(r0 + j * RST, 8)
            pltpu.sync_copy(A.at[pl.ds(o, RST)], wb)
            pltpu.sync_copy(wb, out_hbm.at[cid, pl.ds(o, RST)])
            return carry

        lax.fori_loop(0, RMAIN // RST, obody, 0)

        @pl.when(sid < (N - NSUB * RMAIN) // 8)
        def _write_tail():
            pltpu.sync_copy(A.at[pl.ds(t0, 8)], zbuf)
            pltpu.sync_copy(zbuf, out_hbm.at[cid, pl.ds(t0, 8)])

    return agg


_sc_cache = {}


def _sc_agg80(p, src, dst):
    if W0 not in _sc_cache:
        _sc_cache[W0] = _make_sc_agg(W0)
    return _sc_cache[W0](p, src, dst)


def _pad_table(p):
    ones = jnp.ones((N, 1), jnp.float32)
    zeros = jnp.zeros((N, W0 - H - 1), jnp.float32)
    return jnp.concatenate([p, ones, zeros], axis=1)


def _tc0_body(x_ref, wl_ref, wr_ref, b_ref, p_ref, q_ref):
    x = x_ref[...]
    p_ref[...] = _pad_table(jnp.dot(x, wl_ref[...], precision=_PREC))
    q_ref[...] = jnp.dot(x, wr_ref[...], precision=_PREC) + b_ref[...]


def _tc0(x, wl, wr, b):
    return pl.pallas_call(
        _tc0_body,
        out_shape=(
            jax.ShapeDtypeStruct((N, W0), jnp.float32),
            jax.ShapeDtypeStruct((N, H), jnp.float32),
        ),
    )(x, wl, wr, b)


def _bn_relu(h, g, be):
    mu = jnp.mean(h, axis=0, keepdims=True)
    var = jnp.mean((h - mu) * (h - mu), axis=0, keepdims=True)
    return jnp.maximum(g * (h - mu) / jnp.sqrt(var + EPS) + be, 0.0)


def _tc1_body(agg_ref, q_ref, g_ref, be_ref, wl_ref, wr_ref, b_ref,
              p_ref, qn_ref, cnt_ref):
    a = agg_ref[0] + agg_ref[1]
    cnt = jnp.maximum(a[:, H:H + 1], 1.0)
    h = a[:, :H] / cnt + q_ref[...]
    h = _bn_relu(h, g_ref[...], be_ref[...])
    p_ref[...] = _pad_table(jnp.dot(h, wl_ref[...], precision=_PREC))
    qn_ref[...] = jnp.dot(h, wr_ref[...], precision=_PREC) + b_ref[...]
    cnt_ref[...] = cnt


def _tc1(agg, q, g, be, wl, wr, b):
    return pl.pallas_call(
        _tc1_body,
        out_shape=(
            jax.ShapeDtypeStruct((N, W0), jnp.float32),
            jax.ShapeDtypeStruct((N, H), jnp.float32),
            jax.ShapeDtypeStruct((N, 1), jnp.float32),
        ),
    )(agg, q, g, be, wl, wr, b)


def _tc2_body(agg_ref, cnt_ref, q_ref, g_ref, be_ref, wl_ref, wr_ref, b_ref,
              p_ref, qn_ref):
    a = agg_ref[0] + agg_ref[1]
    h = a[:, :H] / cnt_ref[...] + q_ref[...]
    h = _bn_relu(h, g_ref[...], be_ref[...])
    p_ref[...] = _pad_table(jnp.dot(h, wl_ref[...], precision=_PREC))
    qn_ref[...] = jnp.dot(h, wr_ref[...], precision=_PREC) + b_ref[...]


def _tc2(agg, cnt, q, g, be, wl, wr, b):
    return pl.pallas_call(
        _tc2_body,
        out_shape=(
            jax.ShapeDtypeStruct((N, W0), jnp.float32),
            jax.ShapeDtypeStruct((N, H), jnp.float32),
        ),
    )(agg, cnt, q, g, be, wl, wr, b)


def _tc3_body(agg_ref, cnt_ref, q_ref, g_ref, be_ref, batch_ref,
              wa_ref, ba_ref, wb_ref, bb_ref, out_ref):
    a = agg_ref[0] + agg_ref[1]
    h = a[:, :H] / cnt_ref[...] + q_ref[...]
    h = _bn_relu(h, g_ref[...], be_ref[...])
    gid = lax.broadcasted_iota(jnp.int32, (1, G), 1)
    m = (batch_ref[...] == gid).astype(jnp.float32)
    sums = lax.dot_general(m, h, (((0,), (0,)), ((), ())),
                           precision=_PREC)
    ones = jnp.ones((N, 1), jnp.float32)
    pcnt = lax.dot_general(m, ones, (((0,), (0,)), ((), ())),
                           precision=_PREC)
    emb = sums / jnp.maximum(pcnt, 1.0)
    z = jnp.maximum(jnp.dot(emb, wa_ref[...], precision=_PREC) + ba_ref[...],
                    0.0)
    out_ref[...] = jnp.dot(z, wb_ref[...], precision=_PREC) + bb_ref[...]


def _tc3(agg, cnt, q, g, be, batch2, wa, ba, wb, bb):
    return pl.pallas_call(
        _tc3_body,
        out_shape=jax.ShapeDtypeStruct((G, 2), jnp.float32),
    )(agg, cnt, q, g, be, batch2, wa, ba, wb, bb)


def kernel(x, edge_index, batch, Wl0, Wr0, b0, g0, be0, Wl1, Wr1, b1, g1, be1,
           Wl2, Wr2, b2, g2, be2, Wa, ba, Wb, bb):
    src = edge_index[0].reshape(NW, NCH, CH)
    dst = edge_index[1].reshape(NW, NCH, CH)
    batch2 = batch.reshape(N, 1)
    b0r, g0r, be0r = b0.reshape(1, H), g0.reshape(1, H), be0.reshape(1, H)
    b1r, g1r, be1r = b1.reshape(1, H), g1.reshape(1, H), be1.reshape(1, H)
    b2r, g2r, be2r = b2.reshape(1, H), g2.reshape(1, H), be2.reshape(1, H)
    bar = ba.reshape(1, H // 2)
    bbr = bb.reshape(1, 2)

    p0, q0 = _tc0(x, Wl0, Wr0, b0r)
    agg0 = _sc_agg80(p0, src, dst)
    p1, q1, cnt = _tc1(agg0, q0, g0r, be0r, Wl1, Wr1, b1r)
    agg1 = _sc_agg80(p1, src, dst)
    p2, q2 = _tc2(agg1, cnt, q1, g1r, be1r, Wl2, Wr2, b2r)
    agg2 = _sc_agg80(p2, src, dst)
    out = _tc3(agg2, cnt, q2, g2r, be2r, batch2, Wa, bar, Wb, bbr)
    return out


# speedup vs baseline: 13.9132x; 1.0787x over previous
import functools

import jax
import jax.numpy as jnp
from jax import lax
from jax.experimental import pallas as pl
from jax.experimental.pallas import tpu as pltpu
from jax.experimental.pallas import tpu_sc as plsc

N = 10000
E = 320000
D = 128
H = 64
G = 64
EPS = 1e-5

W0 = 80

NCORES = 2
NSUB = 16
NW = NCORES * NSUB
EPW = E // NW
CH = 125
NCH = EPW // CH
RMAIN = 624
RST = 104

_PREC = lax.Precision.DEFAULT


def _make_sc_agg(W):
    mesh = plsc.VectorSubcoreMesh(core_axis_name="c", subcore_axis_name="s")

    @functools.partial(
        pl.kernel,
        mesh=mesh,
        out_type=jax.ShapeDtypeStruct((NCORES, N, W), jnp.float32),
        scratch_types=[
            pltpu.VMEM((NCH, CH), jnp.int32),
            pltpu.VMEM((NCH, CH), jnp.int32),
            pltpu.VMEM((4, CH, W), jnp.float32),
            pltpu.VMEM((8, W), jnp.float32),
            pltpu.VMEM((RST, W), jnp.float32),
            pltpu.VMEM_SHARED((N, W), jnp.float32),
            [pltpu.SemaphoreType.DMA] * 4,
        ],
        compiler_params=pltpu.CompilerParams(use_tc_tiling_on_sc=False),
    )
    def agg(p_hbm, src_hbm, dst_hbm, out_hbm, sidx, didx, rows,
            zbuf, wb, A, sems):
        cid = lax.axis_index("c")
        sid = lax.axis_index("s")
        wid = cid * NSUB + sid
        r0 = pl.multiple_of(sid * RMAIN, 8)
        t0 = pl.multiple_of(NSUB * RMAIN + sid * 8, 8)

        pltpu.sync_copy(src_hbm.at[wid], sidx)
        pltpu.sync_copy(dst_hbm.at[wid], didx)

        zeros16 = jnp.zeros((16,), jnp.float32)
        for r in range(8):
            for c in range(W // 16):
                zbuf[r, pl.ds(c * 16, 16)] = zeros16

        def zbody(j, carry):
            pltpu.sync_copy(zbuf, A.at[pl.ds(pl.multiple_of(r0 + j * 8, 8), 8)])
            return carry

        lax.fori_loop(0, RMAIN // 8, zbody, 0)

        @pl.when(sid < (N - NSUB * RMAIN) // 8)
        def _zero_tail():
            pltpu.sync_copy(zbuf, A.at[pl.ds(t0, 8)])

        plsc.subcore_barrier()

        for b in range(3):
            pltpu.async_copy(p_hbm.at[sidx.at[b]], rows.at[b], sems[b])

        def ebody(k, carry):
            i0 = 4 * k
            for b in range(4):
                i = i0 + b
                inext = lax.rem(i + 3, NCH)
                pltpu.async_copy(p_hbm.at[sidx.at[inext]],
                                 rows.at[(b + 3) % 4], sems[(b + 3) % 4])
                pltpu.make_async_copy(p_hbm.at[sidx.at[i]],
                                      rows.at[b], sems[b]).wait()
                pltpu.sync_copy(rows.at[b], A.at[didx.at[i]], add=True)
            return carry

        lax.fori_loop(0, NCH // 4, ebody, 0)

        for b in range(3):
            pltpu.make_async_copy(p_hbm.at[sidx.at[b]], rows.at[b],
                                  sems[b]).wait()

        plsc.subcore_barrier()

        def obody(j, carry):
            o = pl.multiple_of(r0 + j * RST, 8)
            pltpu.sync_copy(A.at[pl.ds(o, RST)], wb)
            pltpu.sync_copy(wb, out_hbm.at[cid, pl.ds(o, RST)])
            return carry

        lax.fori_loop(0, RMAIN // RST, obody, 0)

        @pl.when(sid < (N - NSUB * RMAIN) // 8)
        def _write_tail():
            pltpu.sync_copy(A.at[pl.ds(t0, 8)], zbuf)
            pltpu.sync_copy(zbuf, out_hbm.at[cid, pl.ds(t0, 8)])

    return agg


_sc_cache = {}


def _sc_agg80(p, src, dst):
    if W0 not in _sc_cache:
        _sc_cache[W0] = _make_sc_agg(W0)
    return _sc_cache[W0](p, src, dst)


def _pad_table(p):
    ones = jnp.ones((N, 1), jnp.float32)
    zeros = jnp.zeros((N, W0 - H - 1), jnp.float32)
    return jnp.concatenate([p, ones, zeros], axis=1)


def _tc0_body(x_ref, wl_ref, wr_ref, b_ref, p_ref, q_ref):
    x = x_ref[...]
    p_ref[...] = _pad_table(jnp.dot(x, wl_ref[...], precision=_PREC))
    q_ref[...] = jnp.dot(x, wr_ref[...], precision=_PREC) + b_ref[...]


def _tc0(x, wl, wr, b):
    return pl.pallas_call(
        _tc0_body,
        out_shape=(
            jax.ShapeDtypeStruct((N, W0), jnp.float32),
            jax.ShapeDtypeStruct((N, H), jnp.float32),
        ),
    )(x, wl, wr, b)


def _bn_relu(h, g, be):
    mu = jnp.mean(h, axis=0, keepdims=True)
    var = jnp.mean((h - mu) * (h - mu), axis=0, keepdims=True)
    return jnp.maximum(g * (h - mu) / jnp.sqrt(var + EPS) + be, 0.0)


def _tc1_body(agg_ref, q_ref, g_ref, be_ref, wl_ref, wr_ref, b_ref,
              p_ref, qn_ref, cnt_ref):
    a = agg_ref[0] + agg_ref[1]
    cnt = jnp.maximum(a[:, H:H + 1], 1.0)
    h = a[:, :H] / cnt + q_ref[...]
    h = _bn_relu(h, g_ref[...], be_ref[...])
    p_ref[...] = _pad_table(jnp.dot(h, wl_ref[...], precision=_PREC))
    qn_ref[...] = jnp.dot(h, wr_ref[...], precision=_PREC) + b_ref[...]
    cnt_ref[...] = cnt


def _tc1(agg, q, g, be, wl, wr, b):
    return pl.pallas_call(
        _tc1_body,
        out_shape=(
            jax.ShapeDtypeStruct((N, W0), jnp.float32),
            jax.ShapeDtypeStruct((N, H), jnp.float32),
            jax.ShapeDtypeStruct((N, 1), jnp.float32),
        ),
    )(agg, q, g, be, wl, wr, b)


def _tc2_body(agg_ref, cnt_ref, q_ref, g_ref, be_ref, wl_ref, wr_ref, b_ref,
              p_ref, qn_ref):
    a = agg_ref[0] + agg_ref[1]
    h = a[:, :H] / cnt_ref[...] + q_ref[...]
    h = _bn_relu(h, g_ref[...], be_ref[...])
    p_ref[...] = _pad_table(jnp.dot(h, wl_ref[...], precision=_PREC))
    qn_ref[...] = jnp.dot(h, wr_ref[...], precision=_PREC) + b_ref[...]


def _tc2(agg, cnt, q, g, be, wl, wr, b):
    return pl.pallas_call(
        _tc2_body,
        out_shape=(
            jax.ShapeDtypeStruct((N, W0), jnp.float32),
            jax.ShapeDtypeStruct((N, H), jnp.float32),
        ),
    )(agg, cnt, q, g, be, wl, wr, b)


def _tc3_body(agg_ref, cnt_ref, q_ref, g_ref, be_ref, batch_ref,
              wa_ref, ba_ref, wb_ref, bb_ref, out_ref):
    a = agg_ref[0] + agg_ref[1]
    h = a[:, :H] / cnt_ref[...] + q_ref[...]
    h = _bn_relu(h, g_ref[...], be_ref[...])
    gid = lax.broadcasted_iota(jnp.int32, (1, G), 1)
    m = (batch_ref[...] == gid).astype(jnp.float32)
    sums = lax.dot_general(m, h, (((0,), (0,)), ((), ())),
                           precision=_PREC)
    ones = jnp.ones((N, 1), jnp.float32)
    pcnt = lax.dot_general(m, ones, (((0,), (0,)), ((), ())),
                           precision=_PREC)
    emb = sums / jnp.maximum(pcnt, 1.0)
    z = jnp.maximum(jnp.dot(emb, wa_ref[...], precision=_PREC) + ba_ref[...],
                    0.0)
    out_ref[...] = jnp.dot(z, wb_ref[...], precision=_PREC) + bb_ref[...]


def _tc3(agg, cnt, q, g, be, batch2, wa, ba, wb, bb):
    return pl.pallas_call(
        _tc3_body,
        out_shape=jax.ShapeDtypeStruct((G, 2), jnp.float32),
    )(agg, cnt, q, g, be, batch2, wa, ba, wb, bb)


def kernel(x, edge_index, batch, Wl0, Wr0, b0, g0, be0, Wl1, Wr1, b1, g1, be1,
           Wl2, Wr2, b2, g2, be2, Wa, ba, Wb, bb):
    src = edge_index[0].reshape(NW, NCH, CH)
    dst = edge_index[1].reshape(NW, NCH, CH)
    batch2 = batch.reshape(N, 1)
    b0r, g0r, be0r = b0.reshape(1, H), g0.reshape(1, H), be0.reshape(1, H)
    b1r, g1r, be1r = b1.reshape(1, H), g1.reshape(1, H), be1.reshape(1, H)
    b2r, g2r, be2r = b2.reshape(1, H), g2.reshape(1, H), be2.reshape(1, H)
    bar = ba.reshape(1, H // 2)
    bbr = bb.reshape(1, 2)

    p0, q0 = _tc0(x, Wl0, Wr0, b0r)
    agg0 = _sc_agg80(p0, src, dst)
    p1, q1, cnt = _tc1(agg0, q0, g0r, be0r, Wl1, Wr1, b1r)
    agg1 = _sc_agg80(p1, src, dst)
    p2, q2 = _tc2(agg1, cnt, q1, g1r, be1r, Wl2, Wr2, b2r)
    agg2 = _sc_agg80(p2, src, dst)
    out = _tc3(agg2, cnt, q2, g2r, be2r, batch2, Wa, bar, Wb, bbr)
    return out


# speedup vs baseline: 14.7308x; 1.0588x over previous
import functools

import jax
import jax.numpy as jnp
from jax import lax
from jax.experimental import pallas as pl
from jax.experimental.pallas import tpu as pltpu
from jax.experimental.pallas import tpu_sc as plsc

N = 10000
E = 320000
D = 128
H = 64
G = 64
EPS = 1e-5

W0 = 64
CW = 16

NCORES = 2
NSUB = 16
NW = NCORES * NSUB
EPW = E // NW
CH = 125
NCH = EPW // CH
RMAIN = 624
RST = 104

_PREC = lax.Precision.DEFAULT


def _make_sc_agg(W):
    mesh = plsc.VectorSubcoreMesh(core_axis_name="c", subcore_axis_name="s")

    @functools.partial(
        pl.kernel,
        mesh=mesh,
        out_type=jax.ShapeDtypeStruct((NCORES, N, W), jnp.float32),
        scratch_types=[
            pltpu.VMEM((NCH, CH), jnp.int32),
            pltpu.VMEM((NCH, CH), jnp.int32),
            pltpu.VMEM((4, CH, W), jnp.float32),
            pltpu.VMEM((8, W), jnp.float32),
            pltpu.VMEM((RST, W), jnp.float32),
            pltpu.VMEM_SHARED((N, W), jnp.float32),
            [pltpu.SemaphoreType.DMA] * 4,
        ],
        compiler_params=pltpu.CompilerParams(use_tc_tiling_on_sc=False),
    )
    def agg(p_hbm, src_hbm, dst_hbm, out_hbm, sidx, didx, rows,
            zbuf, wb, A, sems):
        cid = lax.axis_index("c")
        sid = lax.axis_index("s")
        wid = cid * NSUB + sid
        r0 = pl.multiple_of(sid * RMAIN, 8)
        t0 = pl.multiple_of(NSUB * RMAIN + sid * 8, 8)

        pltpu.sync_copy(src_hbm.at[wid], sidx)
        pltpu.sync_copy(dst_hbm.at[wid], didx)

        zeros16 = jnp.zeros((16,), jnp.float32)
        for r in range(8):
            for c in range(W // 16):
                zbuf[r, pl.ds(c * 16, 16)] = zeros16

        def zbody(j, carry):
            pltpu.sync_copy(zbuf, A.at[pl.ds(pl.multiple_of(r0 + j * 8, 8), 8)])
            return carry

        lax.fori_loop(0, RMAIN // 8, zbody, 0)

        @pl.when(sid < (N - NSUB * RMAIN) // 8)
        def _zero_tail():
            pltpu.sync_copy(zbuf, A.at[pl.ds(t0, 8)])

        plsc.subcore_barrier()

        for b in range(3):
            pltpu.async_copy(p_hbm.at[sidx.at[b]], rows.at[b], sems[b])

        def ebody(k, carry):
            i0 = 4 * k
            for b in range(4):
                i = i0 + b
                inext = lax.rem(i + 3, NCH)
                pltpu.async_copy(p_hbm.at[sidx.at[inext]],
                                 rows.at[(b + 3) % 4], sems[(b + 3) % 4])
                pltpu.make_async_copy(p_hbm.at[sidx.at[i]],
                                      rows.at[b], sems[b]).wait()
                pltpu.sync_copy(rows.at[b], A.at[didx.at[i]], add=True)
            return carry

        lax.fori_loop(0, NCH // 4, ebody, 0)

        for b in range(3):
            pltpu.make_async_copy(p_hbm.at[sidx.at[b]], rows.at[b],
                                  sems[b]).wait()

        plsc.subcore_barrier()

        def obody(j, carry):
            o = pl.multiple_of(r0 + j * RST, 8)
            pltpu.sync_copy(A.at[pl.ds(o, RST)], wb)
            pltpu.sync_copy(wb, out_hbm.at[cid, pl.ds(o, RST)])
            return carry

        lax.fori_loop(0, RMAIN // RST, obody, 0)

        @pl.when(sid < (N - NSUB * RMAIN) // 8)
        def _write_tail():
            pltpu.sync_copy(A.at[pl.ds(t0, 8)], zbuf)
            pltpu.sync_copy(zbuf, out_hbm.at[cid, pl.ds(t0, 8)])

    return agg


_sc_cache = {}


def _sc_agg80(p, src, dst):
    if W0 not in _sc_cache:
        _sc_cache[W0] = _make_sc_agg(W0)
    return _sc_cache[W0](p, src, dst)


def _make_sc_counts():
    mesh = plsc.VectorSubcoreMesh(core_axis_name="c", subcore_axis_name="s")

    @functools.partial(
        pl.kernel,
        mesh=mesh,
        out_type=jax.ShapeDtypeStruct((NCORES, N, CW), jnp.float32),
        scratch_types=[
            pltpu.VMEM((NCH, CH), jnp.int32),
            pltpu.VMEM((CH, CW), jnp.float32),
            pltpu.VMEM((8, CW), jnp.float32),
            pltpu.VMEM((RST, CW), jnp.float32),
            pltpu.VMEM_SHARED((N, CW), jnp.float32),
        ],
        compiler_params=pltpu.CompilerParams(use_tc_tiling_on_sc=False),
    )
    def cnts(dst_hbm, out_hbm, didx, ones, zbuf, wb, C):
        cid = lax.axis_index("c")
        sid = lax.axis_index("s")
        wid = cid * NSUB + sid
        r0 = pl.multiple_of(sid * RMAIN, 8)
        t0 = pl.multiple_of(NSUB * RMAIN + sid * 8, 8)

        pltpu.sync_copy(dst_hbm.at[wid], didx)

        zeros16 = jnp.zeros((16,), jnp.float32)
        ones16 = jnp.ones((16,), jnp.float32)
        for r in range(8):
            zbuf[r, pl.ds(0, 16)] = zeros16
        for r in range(CH):
            ones[r, pl.ds(0, 16)] = ones16

        def zbody(j, carry):
            pltpu.sync_copy(zbuf, C.at[pl.ds(pl.multiple_of(r0 + j * 8, 8), 8)])
            return carry

        lax.fori_loop(0, RMAIN // 8, zbody, 0)

        @pl.when(sid < (N - NSUB * RMAIN) // 8)
        def _zero_tail():
            pltpu.sync_copy(zbuf, C.at[pl.ds(t0, 8)])

        plsc.subcore_barrier()

        def cbody(i, carry):
            pltpu.sync_copy(ones, C.at[didx.at[i]], add=True)
            return carry

        lax.fori_loop(0, NCH, cbody, 0)

        plsc.subcore_barrier()

        def obody(j, carry):
            o = pl.multiple_of(r0 + j * RST, 8)
            pltpu.sync_copy(C.at[pl.ds(o, RST)], wb)
            pltpu.sync_copy(wb, out_hbm.at[cid, pl.ds(o, RST)])
            return carry

        lax.fori_loop(0, RMAIN // RST, obody, 0)

        @pl.when(sid < (N - NSUB * RMAIN) // 8)
        def _write_tail():
            pltpu.sync_copy(C.at[pl.ds(t0, 8)], zbuf)
            pltpu.sync_copy(zbuf, out_hbm.at[cid, pl.ds(t0, 8)])

    return cnts


def _sc_counts(dst):
    if "cnt" not in _sc_cache:
        _sc_cache["cnt"] = _make_sc_counts()
    return _sc_cache["cnt"](dst)


def _tc0_body(x_ref, wl_ref, wr_ref, b_ref, p_ref, q_ref):
    x = x_ref[...]
    p_ref[...] = jnp.dot(x, wl_ref[...], precision=_PREC)
    q_ref[...] = jnp.dot(x, wr_ref[...], precision=_PREC) + b_ref[...]


def _tc0(x, wl, wr, b):
    return pl.pallas_call(
        _tc0_body,
        out_shape=(
            jax.ShapeDtypeStruct((N, W0), jnp.float32),
            jax.ShapeDtypeStruct((N, H), jnp.float32),
        ),
    )(x, wl, wr, b)


def _bn_relu(h, g, be):
    mu = jnp.mean(h, axis=0, keepdims=True)
    var = jnp.mean((h - mu) * (h - mu), axis=0, keepdims=True)
    return jnp.maximum(g * (h - mu) / jnp.sqrt(var + EPS) + be, 0.0)


def _tc1_body(agg_ref, cin_ref, q_ref, g_ref, be_ref, wl_ref, wr_ref, b_ref,
              p_ref, qn_ref, cnt_ref):
    a = agg_ref[0] + agg_ref[1]
    cnt = jnp.maximum(cin_ref[0][:, :1] + cin_ref[1][:, :1], 1.0)
    h = a / cnt + q_ref[...]
    h = _bn_relu(h, g_ref[...], be_ref[...])
    p_ref[...] = jnp.dot(h, wl_ref[...], precision=_PREC)
    qn_ref[...] = jnp.dot(h, wr_ref[...], precision=_PREC) + b_ref[...]
    cnt_ref[...] = cnt


def _tc1(agg, cin, q, g, be, wl, wr, b):
    return pl.pallas_call(
        _tc1_body,
        out_shape=(
            jax.ShapeDtypeStruct((N, W0), jnp.float32),
            jax.ShapeDtypeStruct((N, H), jnp.float32),
            jax.ShapeDtypeStruct((N, 1), jnp.float32),
        ),
    )(agg, cin, q, g, be, wl, wr, b)


def _tc2_body(agg_ref, cnt_ref, q_ref, g_ref, be_ref, wl_ref, wr_ref, b_ref,
              p_ref, qn_ref):
    a = agg_ref[0] + agg_ref[1]
    h = a / cnt_ref[...] + q_ref[...]
    h = _bn_relu(h, g_ref[...], be_ref[...])
    p_ref[...] = jnp.dot(h, wl_ref[...], precision=_PREC)
    qn_ref[...] = jnp.dot(h, wr_ref[...], precision=_PREC) + b_ref[...]


def _tc2(agg, cnt, q, g, be, wl, wr, b):
    return pl.pallas_call(
        _tc2_body,
        out_shape=(
            jax.ShapeDtypeStruct((N, W0), jnp.float32),
            jax.ShapeDtypeStruct((N, H), jnp.float32),
        ),
    )(agg, cnt, q, g, be, wl, wr, b)


def _tc3_body(agg_ref, cnt_ref, q_ref, g_ref, be_ref, batch_ref,
              wa_ref, ba_ref, wb_ref, bb_ref, out_ref):
    a = agg_ref[0] + agg_ref[1]
    h = a / cnt_ref[...] + q_ref[...]
    h = _bn_relu(h, g_ref[...], be_ref[...])
    gid = lax.broadcasted_iota(jnp.int32, (1, G), 1)
    m = (batch_ref[...] == gid).astype(jnp.float32)
    sums = lax.dot_general(m, h, (((0,), (0,)), ((), ())),
                           precision=_PREC)
    ones = jnp.ones((N, 1), jnp.float32)
    pcnt = lax.dot_general(m, ones, (((0,), (0,)), ((), ())),
                           precision=_PREC)
    emb = sums / jnp.maximum(pcnt, 1.0)
    z = jnp.maximum(jnp.dot(emb, wa_ref[...], precision=_PREC) + ba_ref[...],
                    0.0)
    out_ref[...] = jnp.dot(z, wb_ref[...], precision=_PREC) + bb_ref[...]


def _tc3(agg, cnt, q, g, be, batch2, wa, ba, wb, bb):
    return pl.pallas_call(
        _tc3_body,
        out_shape=jax.ShapeDtypeStruct((G, 2), jnp.float32),
    )(agg, cnt, q, g, be, batch2, wa, ba, wb, bb)


def kernel(x, edge_index, batch, Wl0, Wr0, b0, g0, be0, Wl1, Wr1, b1, g1, be1,
           Wl2, Wr2, b2, g2, be2, Wa, ba, Wb, bb):
    src = edge_index[0].reshape(NW, NCH, CH)
    dst = edge_index[1].reshape(NW, NCH, CH)
    batch2 = batch.reshape(N, 1)
    b0r, g0r, be0r = b0.reshape(1, H), g0.reshape(1, H), be0.reshape(1, H)
    b1r, g1r, be1r = b1.reshape(1, H), g1.reshape(1, H), be1.reshape(1, H)
    b2r, g2r, be2r = b2.reshape(1, H), g2.reshape(1, H), be2.reshape(1, H)
    bar = ba.reshape(1, H // 2)
    bbr = bb.reshape(1, 2)

    p0, q0 = _tc0(x, Wl0, Wr0, b0r)
    cnts = _sc_counts(dst)
    agg0 = _sc_agg80(p0, src, dst)
    p1, q1, cnt = _tc1(agg0, cnts, q0, g0r, be0r, Wl1, Wr1, b1r)
    agg1 = _sc_agg80(p1, src, dst)
    p2, q2 = _tc2(agg1, cnt, q1, g1r, be1r, Wl2, Wr2, b2r)
    agg2 = _sc_agg80(p2, src, dst)
    out = _tc3(agg2, cnt, q2, g2r, be2r, batch2, Wa, bar, Wb, bbr)
    return out
